# Initial kernel scaffold; baseline (speedup 1.0000x reference)
import jax
import jax.numpy as jnp
from jax.experimental import pallas as pl


def kernel(mask, R, T, focal, principal_point):
    raise NotImplementedError("write your pallas kernel here")


# speedup vs baseline: 1.8944x; 1.8944x over previous
import functools
import jax
import jax.numpy as jnp
from jax import lax
from jax.experimental import pallas as pl
from jax.experimental.pallas import tpu as pltpu

IMAGE_H = 400
IMAGE_W = 400
N_RAYS = 1024
N_PTS = 64
MIN_DEPTH = 0.1
MAX_DEPTH = 10.0


def _body(mask_hbm, u_ref, R_ref, p_ref, o_ref, x_vmem, dma_sem):
    b = pl.program_id(0)
    cp = pltpu.make_async_copy(
        mask_hbm.at[b, :, pl.ds(0, 896)], x_vmem, dma_sem
    )
    cp.start()
    cp.wait()
    x = x_vmem[...]
    lane = lax.broadcasted_iota(jnp.int32, (IMAGE_H, 896), 1)
    xm = jnp.where(((lane % 2) == 0) & (lane < 800), x, 0.0)

    c = xm
    s = 1
    while s < 896:
        shifted = jnp.concatenate(
            [jnp.zeros((IMAGE_H, s), jnp.float32), c[:, : 896 - s]], axis=1
        )
        c = c + shifted
        s *= 2

    rowsum = c[:, 799:800]
    r_i = lax.broadcasted_iota(jnp.int32, (IMAGE_H, IMAGE_H), 0)
    r_j = lax.broadcasted_iota(jnp.int32, (IMAGE_H, IMAGE_H), 1)
    l_incl = (r_i <= r_j).astype(jnp.float32)
    l_excl = (r_i < r_j).astype(jnp.float32)
    rowcdf_row = lax.dot_general(
        rowsum, l_incl, (((0,), (0,)), ((), ())),
        preferred_element_type=jnp.float32, precision=lax.Precision.HIGHEST,
    )
    rowprev_row = lax.dot_general(
        rowsum, l_excl, (((0,), (0,)), ((), ())),
        preferred_element_type=jnp.float32, precision=lax.Precision.HIGHEST,
    )
    total = rowcdf_row[0, IMAGE_H - 1]

    u = u_ref[0]
    v = u * total

    row_cnt = jnp.sum((rowcdf_row < v).astype(jnp.float32), axis=1, keepdims=True)
    row = jnp.clip(row_cnt, 0.0, IMAGE_H - 1.0)

    cg = c + jnp.transpose(rowprev_row)
    lane400 = lax.broadcasted_iota(jnp.int32, (N_RAYS, IMAGE_H), 1)
    onehot = (lane400 == row.astype(jnp.int32)).astype(jnp.float32)
    g = lax.dot_general(
        onehot, cg, (((1,), (0,)), ((), ())),
        preferred_element_type=jnp.float32, precision=lax.Precision.HIGHEST,
    )
    cnt800 = jnp.sum((g < v).astype(jnp.float32), axis=1, keepdims=True)
    col = jnp.clip(jnp.floor(cnt800 * 0.5), 0.0, IMAGE_W - 1.0)

    half_x = 1.0 / IMAGE_W
    half_y = 1.0 / IMAGE_H
    step_x = jnp.float32((-1.0 + half_x - (1.0 - half_x)) / (IMAGE_W - 1))
    step_y = jnp.float32((-1.0 + half_y - (1.0 - half_y)) / (IMAGE_H - 1))
    xx = jnp.float32(1.0 - half_x) + col * step_x
    yy = jnp.float32(1.0 - half_y) + row * step_y

    fx = p_ref[0, 0, 3]
    fy = p_ref[0, 0, 4]
    ppx = p_ref[0, 0, 5]
    ppy = p_ref[0, 0, 6]
    d0 = (xx - ppx) / fx
    d1 = (yy - ppy) / fy
    R00 = R_ref[0, 0, 0]; R01 = R_ref[0, 0, 1]; R02 = R_ref[0, 0, 2]
    R10 = R_ref[0, 1, 0]; R11 = R_ref[0, 1, 1]; R12 = R_ref[0, 1, 2]
    R20 = R_ref[0, 2, 0]; R21 = R_ref[0, 2, 1]; R22 = R_ref[0, 2, 2]
    w0 = d0 * R00 + d1 * R01 + R02
    w1 = d0 * R10 + d1 * R11 + R12
    w2 = d0 * R20 + d1 * R21 + R22
    inv = lax.rsqrt(w0 * w0 + w1 * w1 + w2 * w2)
    dir0 = w0 * inv
    dir1 = w1 * inv
    dir2 = w2 * inv

    t0 = p_ref[0, 0, 0]; t1 = p_ref[0, 0, 1]; t2 = p_ref[0, 0, 2]
    o0 = -(t0 * R00 + t1 * R01 + t2 * R02)
    o1 = -(t0 * R10 + t1 * R11 + t2 * R12)
    o2 = -(t0 * R20 + t1 * R21 + t2 * R22)
    ones = jnp.ones((N_RAYS, 1), jnp.float32)

    d_iota = lax.broadcasted_iota(jnp.int32, (N_RAYS, N_PTS), 1).astype(jnp.float32)
    d_step = jnp.float32((MAX_DEPTH - MIN_DEPTH) / (N_PTS - 1))
    lengths = jnp.float32(MIN_DEPTH) + d_iota * d_step

    out = jnp.concatenate(
        [o0 * ones, o1 * ones, o2 * ones, dir0, dir1, dir2, lengths, xx, yy],
        axis=1,
    )
    o_ref[0] = out


def kernel(mask, R, T, focal, principal_point):
    B = mask.shape[0]
    mr = mask[:, 0].reshape(B, IMAGE_H, 1600)
    u = jax.random.uniform(jax.random.key(42), (B, N_RAYS), dtype=jnp.float32)
    u = u.reshape(B, N_RAYS, 1)
    params = jnp.concatenate(
        [T, focal, principal_point, jnp.zeros((B, 1), jnp.float32)], axis=1
    ).reshape(B, 1, 8)
    grid = (B,)
    out = pl.pallas_call(
        _body,
        grid=grid,
        in_specs=[
            pl.BlockSpec(memory_space=pl.ANY),
            pl.BlockSpec((1, N_RAYS, 1), lambda b: (b, 0, 0)),
            pl.BlockSpec((1, 3, 3), lambda b: (b, 0, 0)),
            pl.BlockSpec((1, 1, 8), lambda b: (b, 0, 0)),
        ],
        out_specs=pl.BlockSpec((1, N_RAYS, 72), lambda b: (b, 0, 0)),
        out_shape=jax.ShapeDtypeStruct((B, N_RAYS, 72), jnp.float32),
        scratch_shapes=[
            pltpu.VMEM((IMAGE_H, 896), jnp.float32),
            pltpu.SemaphoreType.DMA,
        ],
    )(mr, u, R, params)
    return out
